# baseline (speedup 1.0000x reference)
import jax
import jax.numpy as jnp
from jax import lax
from jax.experimental import pallas as pl
from jax.experimental.pallas import tpu as pltpu
from jax.experimental.pallas import tpu_sc as plsc

_B = 16384
_D = 32
_L = 16
_NW = 32
_BPW = _B // _NW
_CH = 128
_NCH = _BPW // _CH
_GPC = _CH // _L


def _body(users_hbm, books_hbm, ut_hbm, bt_hbm, wb_hbm, out_hbm,
          uidx_v, bidx_v, umaj_v, bmaj_v, ubuf, bbuf, w_v, out_v, sem):
    wid = lax.axis_index("s") * 2 + lax.axis_index("c")
    base = wid * _BPW

    pltpu.sync_copy(users_hbm.at[pl.ds(base, _BPW)], uidx_v)
    pltpu.sync_copy(books_hbm.at[pl.ds(base, _BPW)], bidx_v)
    pltpu.sync_copy(wb_hbm, w_v)

    def idx_body(k, carry):
        sl = pl.ds(k * _L, _L)
        umaj_v[sl] = lax.shift_right_logical(uidx_v[sl], 2)
        bmaj_v[sl] = lax.shift_right_logical(bidx_v[sl], 2)
        return carry
    lax.fori_loop(0, _BPW // _L, idx_body, 0)

    handles = {}

    def fire(c):
        sl = pl.ds(c * _CH, _CH)
        buf = c % 2
        handles[c] = (
            pltpu.async_copy(ut_hbm.at[umaj_v.at[sl]], ubuf.at[buf], sem),
            pltpu.async_copy(bt_hbm.at[bmaj_v.at[sl]], bbuf.at[buf], sem),
        )

    wvecs = [w_v[pl.ds(k * _L, _L)] for k in range(4)]
    bias = w_v[pl.ds(2 * _D, _L)][0]
    wu = [wvecs[j // _L][j % _L] for j in range(_D)]
    wk = [wvecs[2 + j // _L][j % _L] for j in range(_D)]
    lane = lax.iota(jnp.int32, _L)

    for c in range(_NCH):
        fire(c)
        for h in handles.pop(c):
            h.wait()
        ub = ubuf.at[c % 2]
        bb = bbuf.at[c % 2]

        def g_body(g, carry, c=c, ub=ub, bb=bb):
            rows = g * _L + lane
            gsl = pl.ds(c * _CH + g * _L, _L)
            ucolb = lax.shift_left(jnp.bitwise_and(uidx_v[gsl], 3), 5)
            bcolb = lax.shift_left(jnp.bitwise_and(bidx_v[gsl], 3), 5)
            acc = jnp.full((_L,), bias, jnp.float32)
            for j in range(_D):
                uv = plsc.load_gather(ub, [rows, ucolb + j])
                bv = plsc.load_gather(bb, [rows, bcolb + j])
                acc = acc + uv * wu[j] + bv * wk[j]
            out_v[gsl] = acc
            return carry

        lax.fori_loop(0, _GPC, g_body, 0)

    pltpu.sync_copy(out_v, out_hbm.at[pl.ds(base, _BPW)])


@jax.jit
def kernel(users, books, user_table, book_table, W, b):
    users = users.astype(jnp.int32)
    books = books.astype(jnp.int32)
    ut = user_table.reshape(-1, 128)
    bt = book_table.reshape(-1, 128)
    wb = jnp.concatenate(
        [W.reshape(-1), jnp.broadcast_to(b.reshape(-1)[0], (16,))]
    ).astype(jnp.float32)

    mesh = plsc.VectorSubcoreMesh(core_axis_name="c", subcore_axis_name="s")
    run = pl.kernel(
        _body,
        out_type=jax.ShapeDtypeStruct((_B,), jnp.float32),
        mesh=mesh,
        compiler_params=pltpu.CompilerParams(needs_layout_passes=False),
        scratch_types=[
            pltpu.VMEM((_BPW,), jnp.int32),
            pltpu.VMEM((_BPW,), jnp.int32),
            pltpu.VMEM((_BPW,), jnp.int32),
            pltpu.VMEM((_BPW,), jnp.int32),
            pltpu.VMEM((2, _CH, 128), jnp.float32),
            pltpu.VMEM((2, _CH, 128), jnp.float32),
            pltpu.VMEM((2 * _D + _L,), jnp.float32),
            pltpu.VMEM((_BPW,), jnp.float32),
            pltpu.SemaphoreType.DMA,
        ],
    )
    out = run(users, books, ut, bt, wb)
    return out.reshape(_B, 1)
